# baseline (speedup 1.0000x reference)
import functools

import jax
import jax.numpy as jnp
from jax import lax
from jax.experimental import pallas as pl
from jax.experimental.pallas import tpu as pltpu
from jax.experimental.pallas import tpu_sc as plsc

_ATOM_SIZE = 512
_N_ATOMS = 8
_N_SAMPLES = 4096
_POS_LEN = _N_SAMPLES - _ATOM_SIZE
_POS_DIM = 11
_NW = 16
_PPW = _POS_LEN // _NW
_CHUNKS = _PPW // 16
_SEG = _N_SAMPLES // _NW


def _rsqrt(s):
    i = lax.bitcast_convert_type(s, jnp.int32)
    i = jnp.int32(0x5F3759DF) - lax.shift_right_logical(i, 1)
    y = lax.bitcast_convert_type(i, jnp.float32)
    half = jnp.float32(0.5) * s
    for _ in range(3):
        y = y * (jnp.float32(1.5) - half * y * y)
    return y


def _bf16_round(x):
    b = lax.bitcast_convert_type(x, jnp.int32)
    odd = lax.shift_right_logical(b, 16) & jnp.int32(1)
    r = (b + jnp.int32(0x7FFF) + odd) & jnp.int32(-65536)
    return lax.bitcast_convert_type(r, jnp.float32)


@functools.partial(
    pl.kernel,
    out_type=(
        jax.ShapeDtypeStruct((16,), jnp.int32),
        jax.ShapeDtypeStruct((_N_SAMPLES,), jnp.float32),
    ),
    mesh=plsc.VectorSubcoreMesh(
        core_axis_name="c", subcore_axis_name="s", num_cores=1
    ),
    compiler_params=pltpu.CompilerParams(needs_layout_passes=False),
    scratch_types=[
        pltpu.VMEM((_POS_DIM, _PPW), jnp.float32),
        pltpu.VMEM((_N_ATOMS, 16), jnp.float32),
        pltpu.VMEM((_N_ATOMS * _ATOM_SIZE,), jnp.float32),
        pltpu.VMEM((32,), jnp.float32),
        pltpu.VMEM((16,), jnp.int32),
        pltpu.VMEM((_NW * 32,), jnp.float32),
        pltpu.VMEM((_SEG,), jnp.float32),
        pltpu.VMEM_SHARED((_NW * 32,), jnp.float32),
        pltpu.SemaphoreType.DMA,
        pltpu.SemaphoreType.DMA,
        pltpu.SemaphoreType.DMA,
        pltpu.SemaphoreType.DMA,
    ],
)
def _sc_kernel(pos_hbm, ind_hbm, atoms_hbm, idx_out, wave_out,
               pos_v, ind_v, atoms_v, stage2, stage_idx,
               lv, out_seg, cand, sem_pos, sem_ind,
               sem_atoms, sem_idx):
    w = lax.axis_index("s")
    iota = lax.broadcasted_iota(jnp.int32, (16,), 0)

    cp_pos = pltpu.async_copy(pos_hbm.at[w], pos_v, sem_pos)
    cp_ind = pltpu.async_copy(ind_hbm, ind_v, sem_ind)
    cp_atoms = pltpu.async_copy(atoms_hbm, atoms_v, sem_atoms)
    cp_ind.wait()

    ind_rows = [ind_v[i] for i in range(_N_ATOMS)]
    svec = jnp.zeros((16,), jnp.float32)
    for i, row in enumerate(ind_rows):
        svec = jnp.where(iota == i, jnp.sum(row * row), svec)
    rvec = _rsqrt(svec)
    ind_s = []
    for i, row in enumerate(ind_rows):
        hat = _bf16_round(row * rvec[i])
        ind_s.append([hat[d] for d in range(_POS_DIM)])

    neg = jnp.full((16,), -3.0e38, jnp.float32)
    zero_i = jnp.zeros((16,), jnp.int32)
    wbase = w * _PPW

    def chunk_body(c, carry):
        bvals = carry[:_N_ATOMS]
        bidxs = carry[_N_ATOMS:]
        p = [pos_v[d, pl.ds(c * 16, 16)] for d in range(_POS_DIM)]
        s = p[0] * p[0]
        for d in range(1, _POS_DIM):
            s = s + p[d] * p[d]
        r = _rsqrt(s)
        ph = [_bf16_round(p[d] * r) for d in range(_POS_DIM)]
        cur = wbase + c * 16 + iota
        nvals, nidxs = [], []
        for i in range(_N_ATOMS):
            sim = ph[0] * ind_s[i][0]
            for d in range(1, _POS_DIM):
                sim = sim + ph[d] * ind_s[i][d]
            upd = sim > bvals[i]
            nvals.append(jnp.where(upd, sim, bvals[i]))
            nidxs.append(jnp.where(upd, cur, bidxs[i]))
        return tuple(nvals) + tuple(nidxs)

    cp_pos.wait()
    carry = lax.fori_loop(
        0, _CHUNKS, chunk_body,
        (neg,) * _N_ATOMS + (zero_i,) * _N_ATOMS,
        unroll=7,
    )
    bvals = carry[:_N_ATOMS]
    bidxs = carry[_N_ATOMS:]

    valvec = jnp.zeros((16,), jnp.float32)
    idxvec = jnp.zeros((16,), jnp.int32)
    big = jnp.full((16,), 2 ** 30, jnp.int32)
    for i in range(_N_ATOMS):
        m = jnp.max(bvals[i])
        mi = jnp.min(jnp.where(bvals[i] == m, bidxs[i], big))
        valvec = jnp.where(iota == i, m, valvec)
        idxvec = jnp.where(iota == i, mi, idxvec)
    stage2[pl.ds(0, 16)] = valvec
    stage2[pl.ds(16, 16)] = lax.bitcast_convert_type(idxvec, jnp.float32)
    coff = pl.multiple_of(w * 32, 8)
    pltpu.sync_copy(stage2, cand.at[pl.ds(coff, 32)])
    plsc.subcore_barrier()

    pltpu.sync_copy(cand, lv)
    bestv = lv[pl.ds(0, 16)]
    besti = lax.bitcast_convert_type(lv[pl.ds(16, 16)], jnp.int32)
    for ww in range(1, _NW):
        v = lv[pl.ds(ww * 32, 16)]
        ii = lax.bitcast_convert_type(lv[pl.ds(ww * 32 + 16, 16)], jnp.int32)
        take = v > bestv
        bestv = jnp.where(take, v, bestv)
        besti = jnp.where(take, ii, besti)
    final_idx = [besti[i] for i in range(_N_ATOMS)]

    stage_idx[...] = jnp.where(iota < _N_ATOMS, besti, 0)

    @pl.when(w == 0)
    def _():
        pltpu.async_copy(stage_idx, idx_out, sem_idx)

    cp_atoms.wait()
    seg_start = w * _SEG
    zero_f = jnp.zeros((16,), jnp.float32)
    for c in range(_SEG // 16):
        cur = seg_start + c * 16 + iota
        accv = zero_f
        for i in range(_N_ATOMS):
            t = cur - final_idx[i]
            inside = (t >= 0) & (t < _ATOM_SIZE)
            tc = jnp.clip(t, 0, _ATOM_SIZE - 1) + i * _ATOM_SIZE
            v = plsc.load_gather(atoms_v, [tc])
            accv = accv + jnp.where(inside, v, zero_f)
        out_seg[pl.ds(c * 16, 16)] = accv
    hb_start = pl.multiple_of(seg_start, 8)
    pltpu.sync_copy(out_seg, wave_out.at[pl.ds(hb_start, _SEG)])

    @pl.when(w == 0)
    def _():
        pltpu.make_async_copy(stage_idx, idx_out, sem_idx).wait()


def kernel(x, indices, atoms, positions):
    ind = jnp.pad(indices[0], ((0, 0), (0, 16 - _POS_DIM)))
    pos_t = positions.T.reshape(_POS_DIM, _NW, _PPW).transpose(1, 0, 2)
    idx16, wave = _sc_kernel(pos_t, ind, atoms.reshape(-1))
    int_index = idx16[:_N_ATOMS].reshape(1, _N_ATOMS)
    return (int_index, wave.reshape(1, _N_SAMPLES), indices)


# speedup vs baseline: 1.0343x; 1.0343x over previous
import functools

import jax
import jax.numpy as jnp
from jax import lax
from jax.experimental import pallas as pl
from jax.experimental.pallas import tpu as pltpu
from jax.experimental.pallas import tpu_sc as plsc

_ATOM_SIZE = 512
_N_ATOMS = 8
_N_SAMPLES = 4096
_POS_LEN = _N_SAMPLES - _ATOM_SIZE
_POS_DIM = 11
_NW = 16
_PPW = _POS_LEN // _NW
_CHUNKS = _PPW // 16
_SEG = _N_SAMPLES // _NW


def _rsqrt(s):
    i = lax.bitcast_convert_type(s, jnp.int32)
    i = jnp.int32(0x5F3759DF) - lax.shift_right_logical(i, 1)
    y = lax.bitcast_convert_type(i, jnp.float32)
    half = jnp.float32(0.5) * s
    for _ in range(3):
        y = y * (jnp.float32(1.5) - half * y * y)
    return y


def _bf16_round(x):
    b = lax.bitcast_convert_type(x, jnp.int32)
    odd = lax.shift_right_logical(b, 16) & jnp.int32(1)
    r = (b + jnp.int32(0x7FFF) + odd) & jnp.int32(-65536)
    return lax.bitcast_convert_type(r, jnp.float32)


@functools.partial(
    pl.kernel,
    out_type=(
        jax.ShapeDtypeStruct((16,), jnp.int32),
        jax.ShapeDtypeStruct((_N_SAMPLES,), jnp.float32),
    ),
    mesh=plsc.VectorSubcoreMesh(
        core_axis_name="c", subcore_axis_name="s", num_cores=1
    ),
    compiler_params=pltpu.CompilerParams(needs_layout_passes=False),
    scratch_types=[
        pltpu.VMEM((_POS_DIM, _PPW), jnp.float32),
        pltpu.VMEM((_N_ATOMS, 16), jnp.float32),
        pltpu.VMEM((_N_ATOMS * _ATOM_SIZE,), jnp.float32),
        pltpu.VMEM((32,), jnp.float32),
        pltpu.VMEM((16,), jnp.int32),
        pltpu.VMEM((_NW * 32,), jnp.float32),
        pltpu.VMEM((_SEG,), jnp.float32),
        pltpu.VMEM_SHARED((_NW * 32,), jnp.float32),
        pltpu.SemaphoreType.DMA,
        pltpu.SemaphoreType.DMA,
        pltpu.SemaphoreType.DMA,
        pltpu.SemaphoreType.DMA,
    ],
)
def _sc_kernel(pos_hbm, ind_hbm, atoms_hbm, idx_out, wave_out,
               pos_v, ind_v, atoms_v, stage2, stage_idx,
               lv, out_seg, cand, sem_pos, sem_ind,
               sem_atoms, sem_idx):
    w = lax.axis_index("s")
    iota = lax.broadcasted_iota(jnp.int32, (16,), 0)

    cp_pos = pltpu.async_copy(pos_hbm.at[w], pos_v, sem_pos)
    cp_ind = pltpu.async_copy(ind_hbm, ind_v, sem_ind)
    cp_atoms = pltpu.async_copy(atoms_hbm, atoms_v, sem_atoms)
    cp_ind.wait()

    ind_rows = [ind_v[i] for i in range(_N_ATOMS)]
    svec = jnp.zeros((16,), jnp.float32)
    for i, row in enumerate(ind_rows):
        svec = jnp.where(iota == i, jnp.sum(row * row), svec)
    rvec = _rsqrt(svec)
    ind_s = []
    for i, row in enumerate(ind_rows):
        hat = _bf16_round(row * rvec[i])
        ind_s.append([hat[d] for d in range(_POS_DIM)])

    neg = jnp.full((16,), -3.0e38, jnp.float32)
    zero_i = jnp.zeros((16,), jnp.int32)
    wbase = w * _PPW

    def chunk_body(c, carry):
        bvals = carry[:_N_ATOMS]
        bidxs = carry[_N_ATOMS:]
        p = [pos_v[d, pl.ds(c * 16, 16)] for d in range(_POS_DIM)]
        s = p[0] * p[0]
        for d in range(1, _POS_DIM):
            s = s + p[d] * p[d]
        r = _rsqrt(s)
        ph = [_bf16_round(p[d] * r) for d in range(_POS_DIM)]
        cur = wbase + c * 16 + iota
        nvals, nidxs = [], []
        for i in range(_N_ATOMS):
            sim = ph[0] * ind_s[i][0]
            for d in range(1, _POS_DIM):
                sim = sim + ph[d] * ind_s[i][d]
            upd = sim > bvals[i]
            nvals.append(jnp.where(upd, sim, bvals[i]))
            nidxs.append(jnp.where(upd, cur, bidxs[i]))
        return tuple(nvals) + tuple(nidxs)

    cp_pos.wait()
    carry = lax.fori_loop(
        0, _CHUNKS, chunk_body,
        (neg,) * _N_ATOMS + (zero_i,) * _N_ATOMS,
        unroll=2,
    )
    bvals = carry[:_N_ATOMS]
    bidxs = carry[_N_ATOMS:]

    valvec = jnp.zeros((16,), jnp.float32)
    idxvec = jnp.zeros((16,), jnp.int32)
    big = jnp.full((16,), 2 ** 30, jnp.int32)
    for i in range(_N_ATOMS):
        m = jnp.max(bvals[i])
        mi = jnp.min(jnp.where(bvals[i] == m, bidxs[i], big))
        valvec = jnp.where(iota == i, m, valvec)
        idxvec = jnp.where(iota == i, mi, idxvec)
    stage2[pl.ds(0, 16)] = valvec
    stage2[pl.ds(16, 16)] = lax.bitcast_convert_type(idxvec, jnp.float32)
    coff = pl.multiple_of(w * 32, 8)
    pltpu.sync_copy(stage2, cand.at[pl.ds(coff, 32)])
    plsc.subcore_barrier()

    pltpu.sync_copy(cand, lv)
    bestv = lv[pl.ds(0, 16)]
    besti = lax.bitcast_convert_type(lv[pl.ds(16, 16)], jnp.int32)
    for ww in range(1, _NW):
        v = lv[pl.ds(ww * 32, 16)]
        ii = lax.bitcast_convert_type(lv[pl.ds(ww * 32 + 16, 16)], jnp.int32)
        take = v > bestv
        bestv = jnp.where(take, v, bestv)
        besti = jnp.where(take, ii, besti)
    final_idx = [besti[i] for i in range(_N_ATOMS)]

    stage_idx[...] = jnp.where(iota < _N_ATOMS, besti, 0)

    @pl.when(w == 0)
    def _():
        pltpu.async_copy(stage_idx, idx_out, sem_idx)

    cp_atoms.wait()
    seg_start = w * _SEG
    zero_f = jnp.zeros((16,), jnp.float32)

    def place_body(c, _):
        cur = seg_start + c * 16 + iota
        accv = zero_f
        for i in range(_N_ATOMS):
            t = cur - final_idx[i]
            inside = (t >= 0) & (t < _ATOM_SIZE)
            tc = jnp.clip(t, 0, _ATOM_SIZE - 1) + i * _ATOM_SIZE
            v = plsc.load_gather(atoms_v, [tc])
            accv = accv + jnp.where(inside, v, zero_f)
        out_seg[pl.ds(c * 16, 16)] = accv
        return 0

    lax.fori_loop(0, _SEG // 16, place_body, 0, unroll=2)
    hb_start = pl.multiple_of(seg_start, 8)
    pltpu.sync_copy(out_seg, wave_out.at[pl.ds(hb_start, _SEG)])

    @pl.when(w == 0)
    def _():
        pltpu.make_async_copy(stage_idx, idx_out, sem_idx).wait()


def kernel(x, indices, atoms, positions):
    ind = jnp.pad(indices[0], ((0, 0), (0, 16 - _POS_DIM)))
    pos_t = positions.T.reshape(_POS_DIM, _NW, _PPW).transpose(1, 0, 2)
    idx16, wave = _sc_kernel(pos_t, ind, atoms.reshape(-1))
    int_index = idx16[:_N_ATOMS].reshape(1, _N_ATOMS)
    return (int_index, wave.reshape(1, _N_SAMPLES), indices)
